# baseline (speedup 1.0000x reference)
import functools

import jax
import jax.numpy as jnp
from jax import lax
from jax.experimental import pallas as pl
from jax.experimental.pallas import tpu as pltpu
from jax.experimental.pallas import tpu_sc as plsc

VOCAB = 1000000
EMBED = 64
B = 16384
L = 50

NC = 2
NS = 16
NW = NC * NS
BPW = B // NW
CB = 16
NCHUNK = BPW // CB

_sc_mesh = plsc.VectorSubcoreMesh(core_axis_name="c", subcore_axis_name="s")


@functools.partial(
    pl.kernel,
    mesh=_sc_mesh,
    out_type=jax.ShapeDtypeStruct((B, EMBED), jnp.float32),
    scratch_types=[
        pltpu.VMEM((L, CB), jnp.int32),
        pltpu.VMEM((L * CB, 128), jnp.float32),
        pltpu.VMEM((CB, EMBED), jnp.float32),
        pltpu.SemaphoreType.DMA,
    ],
    compiler_params=pltpu.CompilerParams(use_tc_tiling_on_sc=False),
)
def _sc_pool(ids_hbm, table_hbm, out_hbm, ids_v, rows_v, pooled_v, sem):
    wid = lax.axis_index("s") * NC + lax.axis_index("c")

    def chunk_body(ci, carry):
        base = wid * BPW + ci * CB
        pltpu.sync_copy(
            ids_hbm.at[pl.ds(0, L), base // 128, pl.ds(base % 128, CB)],
            ids_v)
        descs = []
        for l in range(L):
            descs.append(pltpu.async_copy(
                table_hbm.at[ids_v.at[l]],
                rows_v.at[pl.ds(l * CB, CB)],
                sem,
            ))
        for d in descs:
            d.wait()

        def b_body(bi, c2):
            for col in range(EMBED // 16):
                acc = rows_v[bi, pl.ds(col * 16, 16)]
                for l in range(1, L):
                    acc = acc + rows_v[l * CB + bi, pl.ds(col * 16, 16)]
                pooled_v[bi, pl.ds(col * 16, 16)] = acc
            return c2

        lax.fori_loop(0, CB, b_body, 0, unroll=False)
        pltpu.sync_copy(pooled_v, out_hbm.at[pl.ds(base, CB)])
        return carry

    lax.fori_loop(0, NCHUNK, chunk_body, 0, unroll=False)


def _tc_transpose_ids(x_ref, o_ref):
    o_ref[...] = x_ref[...].T.reshape(L, 8, 128)


def _tc_pad_table(x_ref, o_ref):
    o_ref[...] = jnp.concatenate(
        [x_ref[...], jnp.zeros(x_ref.shape, jnp.float32)], axis=1)


def _tc_proj(x_ref, w_ref, b_ref, o_ref):
    x = x_ref[...] * (1.0 / L)
    y = jnp.dot(x, w_ref[...].T, preferred_element_type=jnp.float32)
    y = y + b_ref[...]
    n = jnp.sqrt(jnp.sum(y * y, axis=-1, keepdims=True))
    o_ref[...] = y / jnp.maximum(n, 1e-12)


def kernel(input_ids, table, W, b):
    ids_t = pl.pallas_call(
        _tc_transpose_ids,
        grid=(16,),
        in_specs=[pl.BlockSpec((B // 16, L), lambda i: (i, 0))],
        out_specs=pl.BlockSpec((L, 8, 128), lambda i: (0, i, 0)),
        out_shape=jax.ShapeDtypeStruct((L, B // 128, 128), jnp.int32),
    )(input_ids)
    table_p = pl.pallas_call(
        _tc_pad_table,
        grid=(125,),
        in_specs=[pl.BlockSpec((VOCAB // 125, EMBED), lambda i: (i, 0))],
        out_specs=pl.BlockSpec((VOCAB // 125, 128), lambda i: (i, 0)),
        out_shape=jax.ShapeDtypeStruct((VOCAB, 128), jnp.float32),
    )(table)
    pooled = _sc_pool(ids_t, table_p)
    out = pl.pallas_call(
        _tc_proj,
        out_shape=jax.ShapeDtypeStruct((B, EMBED), jnp.float32),
    )(pooled, W, b.reshape(1, EMBED))
    return out


# speedup vs baseline: 1.3518x; 1.3518x over previous
import functools

import jax
import jax.numpy as jnp
from jax import lax
from jax.experimental import pallas as pl
from jax.experimental.pallas import tpu as pltpu
from jax.experimental.pallas import tpu_sc as plsc

VOCAB = 1000000
EMBED = 64
B = 16384
L = 50

NC = 2
NS = 16
NW = NC * NS
BPW = B // NW
CB = 16
NCHUNK = BPW // CB

_sc_mesh = plsc.VectorSubcoreMesh(core_axis_name="c", subcore_axis_name="s")


@functools.partial(
    pl.kernel,
    mesh=_sc_mesh,
    out_type=jax.ShapeDtypeStruct((B, EMBED), jnp.float32),
    scratch_types=[
        pltpu.VMEM((L, CB), jnp.int32),
        pltpu.VMEM((L, CB), jnp.int32),
        pltpu.VMEM((L * CB, EMBED), jnp.float32),
        pltpu.VMEM((L * CB, EMBED), jnp.float32),
        pltpu.VMEM((CB, EMBED), jnp.float32),
        pltpu.SemaphoreType.DMA,
        pltpu.SemaphoreType.DMA,
    ],
    compiler_params=pltpu.CompilerParams(use_tc_tiling_on_sc=False),
)
def _sc_pool(ids_hbm, table_hbm, out_hbm, ids0, ids1, rows0, rows1,
             pooled_v, sem0, sem1):
    wid = lax.axis_index("s") * NC + lax.axis_index("c")
    idsb = (ids0, ids1)
    rowsb = (rows0, rows1)
    semb = (sem0, sem1)

    def stage_fire(ci, par):
        base = wid * BPW + ci * CB
        pltpu.sync_copy(
            ids_hbm.at[pl.ds(0, L), base // 128, pl.ds(base % 128, CB)],
            idsb[par])
        for l in range(L):
            pltpu.async_copy(
                table_hbm.at[idsb[par].at[l]],
                rowsb[par].at[pl.ds(l * CB, CB)],
                semb[par],
            )

    stage_fire(0, 0)

    def pair_body(p, carry):
        for par in range(2):
            ci = p * 2 + par
            nci = ci + 1

            @pl.when(nci < NCHUNK)
            def _():
                stage_fire(nci, 1 - par)

            pltpu.make_async_copy(
                table_hbm.at[pl.ds(0, L * CB)], rowsb[par], semb[par]).wait()

            rows_v = rowsb[par]
            base = wid * BPW + ci * CB

            def b_body(bi, c2):
                for col in range(EMBED // 16):
                    acc = rows_v[bi, pl.ds(col * 16, 16)]
                    for l in range(1, L):
                        acc = acc + rows_v[l * CB + bi, pl.ds(col * 16, 16)]
                    pooled_v[bi, pl.ds(col * 16, 16)] = acc
                return c2

            lax.fori_loop(0, CB, b_body, 0, unroll=False)
            pltpu.sync_copy(pooled_v, out_hbm.at[pl.ds(base, CB)])
        return carry

    lax.fori_loop(0, NCHUNK // 2, pair_body, 0, unroll=False)


def _tc_transpose_ids(x_ref, o_ref):
    o_ref[...] = x_ref[...].T.reshape(L, 8, 128)


def _tc_proj(x_ref, w_ref, b_ref, o_ref):
    x = x_ref[...] * (1.0 / L)
    y = jnp.dot(x, w_ref[...].T, preferred_element_type=jnp.float32)
    y = y + b_ref[...]
    n = jnp.sqrt(jnp.sum(y * y, axis=-1, keepdims=True))
    o_ref[...] = y / jnp.maximum(n, 1e-12)


def kernel(input_ids, table, W, b):
    ids_t = pl.pallas_call(
        _tc_transpose_ids,
        grid=(16,),
        in_specs=[pl.BlockSpec((B // 16, L), lambda i: (i, 0))],
        out_specs=pl.BlockSpec((L, 8, 128), lambda i: (0, i, 0)),
        out_shape=jax.ShapeDtypeStruct((L, B // 128, 128), jnp.int32),
    )(input_ids)
    pooled = _sc_pool(ids_t, table)
    out = pl.pallas_call(
        _tc_proj,
        out_shape=jax.ShapeDtypeStruct((B, EMBED), jnp.float32),
    )(pooled, W, b.reshape(1, EMBED))
    return out
